# baseline (speedup 1.0000x reference)
import functools

import jax
import jax.numpy as jnp
from jax import lax
from jax.experimental import pallas as pl
from jax.experimental.pallas import tpu as pltpu
from jax.experimental.pallas import tpu_sc as plsc

_PLACEHOLDER = -1


def _sc_gather(ids_flat, logits_flat, probs_flat, V):
    R = ids_flat.shape[0]
    info = plsc.get_sparse_core_info()
    nw = info.num_cores * info.num_subcores
    per = R // nw
    mesh = plsc.VectorSubcoreMesh(core_axis_name="c", subcore_axis_name="s")

    @functools.partial(
        pl.kernel,
        mesh=mesh,
        out_type=(
            jax.ShapeDtypeStruct((R,), jnp.float32),
            jax.ShapeDtypeStruct((R,), jnp.float32),
        ),
        scratch_types=[
            pltpu.VMEM((per,), jnp.int32),
            pltpu.VMEM((per,), jnp.int32),
            pltpu.VMEM((per,), jnp.float32),
            pltpu.VMEM((per,), jnp.float32),
            pltpu.SemaphoreType.DMA,
            pltpu.SemaphoreType.DMA,
        ],
    )
    def k(ids_hbm, lf_hbm, pf_hbm, lat_hbm, dat_hbm,
          idx_v, flat_v, lat_v, dat_v, sem1, sem2):
        wid = lax.axis_index("s") * info.num_cores + lax.axis_index("c")
        base = wid * per
        pltpu.sync_copy(ids_hbm.at[pl.ds(base, per)], idx_v)
        rows = lax.iota(jnp.int32, per) + base
        flat_v[...] = rows * V + idx_v[...]
        cp1 = pltpu.async_copy(lf_hbm.at[flat_v], lat_v, sem1)
        cp2 = pltpu.async_copy(pf_hbm.at[flat_v], dat_v, sem2)
        cp1.wait()
        cp2.wait()
        pltpu.sync_copy(lat_v, lat_hbm.at[pl.ds(base, per)])
        pltpu.sync_copy(dat_v, dat_hbm.at[pl.ds(base, per)])

    return k(ids_flat, logits_flat, probs_flat)


def _tc_stats(logits_r):
    G, E, V = logits_r.shape

    def body(lref, mref, zref, aref):
        L = lref[...]
        m = jnp.max(L, axis=2)
        z = jnp.sum(jnp.exp(L - m[:, :, None]), axis=2)
        iota = lax.broadcasted_iota(jnp.int32, L.shape, 2)
        am = jnp.min(jnp.where(L == m[:, :, None], iota, V), axis=2)
        mref[...] = m[None]
        zref[...] = z[None]
        aref[...] = am[None]

    return pl.pallas_call(
        body,
        grid=(G,),
        in_specs=[pl.BlockSpec((1, E, V), lambda i: (i, 0, 0))],
        out_specs=[
            pl.BlockSpec((1, 1, E), lambda i: (i, 0, 0)),
            pl.BlockSpec((1, 1, E), lambda i: (i, 0, 0)),
            pl.BlockSpec((1, 1, E), lambda i: (i, 0, 0)),
        ],
        out_shape=[
            jax.ShapeDtypeStruct((G, 1, E), jnp.float32),
            jax.ShapeDtypeStruct((G, 1, E), jnp.float32),
            jax.ShapeDtypeStruct((G, 1, E), jnp.int32),
        ],
    )(logits_r)


def _tc_scan(m_t, z_t, am_t, ids_t, lat_t, dat_t, u_t, bonus_r, greedy_r):
    S, B = ids_t.shape

    def body(mref, zref, aref, iref, lref, dref, uref, bref, gref,
             outref, nrref, laref, ssref, wrref, msref, zsref):
        m = mref[...]
        z = zref[...]
        am = aref[...]
        ids = iref[...]
        lat = lref[...]
        dat = dref[...]
        u = uref[...]
        bonus = bref[...]
        greedy = gref[...] != 0

        t = jnp.exp(lat - m) / z
        acc = (dat > 0.0) & ((t / jnp.where(dat > 0.0, dat, 1.0)) >= u)
        match = ids == am

        ones = jnp.ones((1, B), dtype=jnp.bool_)
        prev_g = ones
        prev_r = ones
        numacc_g = jnp.zeros((1, B), dtype=jnp.int32)
        numacc_r = jnp.zeros((1, B), dtype=jnp.int32)
        neg1 = jnp.full((1, B), _PLACEHOLDER, dtype=jnp.int32)
        for s in range(S):
            acc_s = acc[s:s + 1, :]
            match_s = match[s:s + 1, :]
            am_s = am[s:s + 1, :]
            ids_s = ids[s:s + 1, :]
            tok_g = jnp.where(prev_g, am_s, neg1)
            tok_r = jnp.where(prev_r, jnp.where(acc_s, ids_s, 0), neg1)
            outref[s:s + 1, :] = jnp.where(greedy, tok_g, tok_r)
            numacc_g += jnp.where(prev_g, 1, 0)
            numacc_r += jnp.where(prev_r, 1, 0)
            prev_g = prev_g & match_s
            prev_r = prev_r & acc_s
        all_g = prev_g
        all_r = prev_r
        numacc_g += jnp.where(all_g, 1, 0)
        numacc_r += jnp.where(all_r, 1, 0)
        ok_bonus = (greedy & all_g) | ((~greedy) & all_r)
        outref[S:S + 1, :] = jnp.where(ok_bonus, bonus, neg1)

        numacc = jnp.where(greedy, numacc_g, numacc_r)
        nrref[...] = (S + 1) - numacc

        first_rj = numacc_r - 1
        sstar = jnp.minimum(first_rj, S - 1)
        ssref[...] = sstar
        wrref[...] = jnp.where((~greedy) & (first_rj < S), 1, 0)

        last_g = bonus
        for s in reversed(range(S)):
            last_g = jnp.where(match[s:s + 1, :], last_g, am[s:s + 1, :])
        laref[...] = jnp.where(greedy, last_g, bonus)

        msel = m[0:1, :]
        zsel = z[0:1, :]
        for s in range(1, S):
            pick = sstar == s
            msel = jnp.where(pick, m[s:s + 1, :], msel)
            zsel = jnp.where(pick, z[s:s + 1, :], zsel)
        msref[...] = msel
        zsref[...] = zsel

    return pl.pallas_call(
        body,
        out_shape=[
            jax.ShapeDtypeStruct((S + 1, B), jnp.int32),
            jax.ShapeDtypeStruct((1, B), jnp.int32),
            jax.ShapeDtypeStruct((1, B), jnp.int32),
            jax.ShapeDtypeStruct((1, B), jnp.int32),
            jax.ShapeDtypeStruct((1, B), jnp.int32),
            jax.ShapeDtypeStruct((1, B), jnp.float32),
            jax.ShapeDtypeStruct((1, B), jnp.float32),
        ],
    )(m_t, z_t, am_t, ids_t, lat_t, dat_t, u_t, bonus_r, greedy_r)


def _tc_recover(sstar, logits, dprobs, q, m_sel, z_sel, wrec, lasta, outa):
    B, S, V = logits.shape
    logits_r = logits.reshape(B * S, 1, V)
    dprobs_r = dprobs.reshape(B * S, 1, V)
    q_r = q.reshape(B, 1, V)
    outa_r = outa.reshape(B, 1, S + 1)

    def body(ss_ref, lref, dref, qref, mref, zref, wref, laref, oaref,
             out_ref, last_ref):
        b = pl.program_id(0)
        m = mref[0, b]
        z = zref[0, b]
        p = jnp.exp(lref[0] - m) / z
        score = jnp.maximum(p - dref[0], 0.0) * (1.0 / qref[0])
        iota = lax.broadcasted_iota(jnp.int32, score.shape, 1)
        smax = jnp.max(score)
        rec = jnp.min(jnp.where(score == smax, iota, V))
        wr = wref[0, b] != 0
        srow = ss_ref[b]
        row = oaref[0]
        io = lax.broadcasted_iota(jnp.int32, row.shape, 1)
        out_ref[...] = jnp.where((io == srow) & wr, rec, row)[None]
        last_ref[...] = jnp.broadcast_to(
            jnp.where(wr, rec, laref[0, b]), (1, 1, 1))

    grid_spec = pltpu.PrefetchScalarGridSpec(
        num_scalar_prefetch=1,
        grid=(B,),
        in_specs=[
            pl.BlockSpec((1, 1, V), lambda b, ss: (b * S + ss[b], 0, 0)),
            pl.BlockSpec((1, 1, V), lambda b, ss: (b * S + ss[b], 0, 0)),
            pl.BlockSpec((1, 1, V), lambda b, ss: (b, 0, 0)),
            pl.BlockSpec(memory_space=pltpu.SMEM),
            pl.BlockSpec(memory_space=pltpu.SMEM),
            pl.BlockSpec(memory_space=pltpu.SMEM),
            pl.BlockSpec(memory_space=pltpu.SMEM),
            pl.BlockSpec((1, 1, S + 1), lambda b, ss: (b, 0, 0)),
        ],
        out_specs=[
            pl.BlockSpec((1, 1, S + 1), lambda b, ss: (b, 0, 0)),
            pl.BlockSpec((1, 1, 1), lambda b, ss: (b, 0, 0)),
        ],
    )
    out, last = pl.pallas_call(
        body,
        grid_spec=grid_spec,
        out_shape=[
            jax.ShapeDtypeStruct((B, 1, S + 1), jnp.int32),
            jax.ShapeDtypeStruct((B, 1, 1), jnp.int32),
        ],
    )(sstar, logits_r, dprobs_r, q_r, m_sel, z_sel, wrec, lasta, outa_r)
    return out.reshape(B, S + 1), last.reshape(B, 1)


def kernel(target_logits, draft_token_ids, bonus_token_ids, is_greedy,
           uniform_probs, q, draft_probs):
    B, S = draft_token_ids.shape
    V = target_logits.shape[-1]
    idt = draft_token_ids.dtype

    ids_flat = draft_token_ids.reshape(-1).astype(jnp.int32)
    logits_flat = target_logits.reshape(-1)
    probs_flat = draft_probs.reshape(-1)
    lat_flat, dat_flat = _sc_gather(ids_flat, logits_flat, probs_flat, V)

    rows8 = (B * S) // 8
    m8, z8, am8 = _tc_stats(target_logits.reshape(rows8, 8, V))

    def t_sb(x):
        return x.reshape(B, S).T

    outa_t, numrej_r, lasta_r, sstar_r, wrec_r, msel_r, zsel_r = _tc_scan(
        t_sb(m8.reshape(-1)), t_sb(z8.reshape(-1)), t_sb(am8.reshape(-1)),
        draft_token_ids.T.astype(jnp.int32), t_sb(lat_flat), t_sb(dat_flat),
        uniform_probs.T,
        bonus_token_ids.reshape(1, B).astype(jnp.int32),
        is_greedy.reshape(1, B).astype(jnp.int32),
    )

    out, last = _tc_recover(
        sstar_r.reshape(B), target_logits, draft_probs, q,
        msel_r, zsel_r, wrec_r, lasta_r, outa_t.T)

    return (out.astype(idt),
            numrej_r.reshape(B).astype(jnp.int32),
            last.reshape(B).astype(idt))


# speedup vs baseline: 1.9333x; 1.9333x over previous
import jax
import jax.numpy as jnp
from jax import lax
from jax.experimental import pallas as pl
from jax.experimental.pallas import tpu as pltpu

_PLACEHOLDER = -1


def _fused(target_logits, draft_probs, q, ids_sm, u_sm, bonus_sm, greedy_sm):
    B, S, V = target_logits.shape

    def body(lref, dpref, qref, idsv_ref, ids_ref, u_ref, bon_ref, grd_ref,
             out_ref, nr_ref, last_ref):
        b = pl.program_id(0)
        greedy = grd_ref[0, b] != 0
        bonus = bon_ref[0, b]

        L = lref[...]
        DP = dpref[...]
        m2 = jnp.max(L, axis=2)
        E = jnp.exp(L - m2[:, :, None])
        z2 = jnp.sum(E, axis=2)
        io3 = lax.broadcasted_iota(jnp.int32, L.shape, 2)
        am2 = jnp.min(jnp.where(L == m2[:, :, None], io3, V), axis=2)
        idsv3 = idsv_ref[...][0][:, :, None]
        hit = io3 == idsv3
        lat2 = jnp.sum(jnp.where(hit, L, 0.0), axis=2)
        dat2 = jnp.sum(jnp.where(hit, DP, 0.0), axis=2)

        ioS = lax.broadcasted_iota(jnp.int32, (1, S), 1)

        def fsel(v, s):
            return jnp.sum(jnp.where(ioS == s, v, 0.0))

        def isel(v, s):
            return jnp.sum(jnp.where(ioS == s, v, 0))

        prev_g = jnp.full((), True)
        prev_r = jnp.full((), True)
        numacc_g = jnp.full((), 0, jnp.int32)
        numacc_r = jnp.full((), 0, jnp.int32)
        toks = []
        ams = []
        matches = []
        ms = []
        zs = []
        for s in range(S):
            m_s = fsel(m2, s)
            z_s = fsel(z2, s)
            am_s = isel(am2, s)
            ms.append(m_s)
            zs.append(z_s)
            ams.append(am_s)
            idx = ids_ref[b, s]
            lat = fsel(lat2, s)
            dat = fsel(dat2, s)
            t = jnp.exp(lat - m_s) / z_s
            acc_s = (dat > 0.0) & (
                (t / jnp.where(dat > 0.0, dat, 1.0)) >= u_ref[b, s])
            match_s = idx == am_s
            matches.append(match_s)
            tok_g = jnp.where(prev_g, am_s, _PLACEHOLDER)
            tok_r = jnp.where(prev_r, jnp.where(acc_s, idx, 0), _PLACEHOLDER)
            toks.append(jnp.where(greedy, tok_g, tok_r))
            numacc_g += jnp.where(prev_g, 1, 0)
            numacc_r += jnp.where(prev_r, 1, 0)
            prev_g = prev_g & match_s
            prev_r = prev_r & acc_s
        all_g = prev_g
        all_r = prev_r
        numacc_g += jnp.where(all_g, 1, 0)
        numacc_r += jnp.where(all_r, 1, 0)
        tok_b = jnp.where((greedy & all_g) | ((~greedy) & all_r),
                          bonus, _PLACEHOLDER)

        numacc = jnp.where(greedy, numacc_g, numacc_r)
        first_rj = numacc_r - 1
        sstar = jnp.minimum(first_rj, S - 1)
        wr = (~greedy) & (first_rj < S)

        last_g = bonus
        for s in reversed(range(S)):
            last_g = jnp.where(matches[s], last_g, ams[s])
        last_nw = jnp.where(greedy, last_g, bonus)

        m_sel = ms[0]
        z_sel = zs[0]
        for s in range(1, S):
            pick = sstar == s
            m_sel = jnp.where(pick, ms[s], m_sel)
            z_sel = jnp.where(pick, zs[s], z_sel)

        ioSub = lax.broadcasted_iota(jnp.int32, (1, S, 1), 1)
        sub = ioSub == sstar
        esel = jnp.sum(jnp.where(sub, E, 0.0), axis=1)
        dsel = jnp.sum(jnp.where(sub, DP, 0.0), axis=1)
        qrow = qref[pl.ds(lax.rem(b, 8), 1), :]
        p = esel / z_sel
        score = jnp.maximum(p - dsel, 0.0) * (1.0 / qrow)
        io2 = lax.broadcasted_iota(jnp.int32, score.shape, 1)
        smax = jnp.max(score)
        rec = jnp.min(jnp.where(score == smax, io2, V))

        io5 = lax.broadcasted_iota(jnp.int32, (1, 1, S + 1), 2)
        row = jnp.broadcast_to(tok_b, (1, 1, S + 1))
        for s in reversed(range(S)):
            tok_f = jnp.where(wr & (sstar == s), rec, toks[s])
            row = jnp.where(io5 == s, tok_f, row)
        out_ref[...] = row
        nr_ref[...] = jnp.broadcast_to((S + 1) - numacc, (1, 1, 1))
        last_ref[...] = jnp.broadcast_to(
            jnp.where(wr, rec, last_nw), (1, 1, 1))

    return pl.pallas_call(
        body,
        grid=(B,),
        in_specs=[
            pl.BlockSpec((1, S, V), lambda b: (b, 0, 0)),
            pl.BlockSpec((1, S, V), lambda b: (b, 0, 0)),
            pl.BlockSpec((8, V), lambda b: (b // 8, 0)),
            pl.BlockSpec((1, 1, S), lambda b: (b, 0, 0)),
            pl.BlockSpec(memory_space=pltpu.SMEM),
            pl.BlockSpec(memory_space=pltpu.SMEM),
            pl.BlockSpec(memory_space=pltpu.SMEM),
            pl.BlockSpec(memory_space=pltpu.SMEM),
        ],
        out_specs=[
            pl.BlockSpec((1, 1, S + 1), lambda b: (b, 0, 0)),
            pl.BlockSpec((1, 1, 1), lambda b: (b, 0, 0)),
            pl.BlockSpec((1, 1, 1), lambda b: (b, 0, 0)),
        ],
        out_shape=[
            jax.ShapeDtypeStruct((B, 1, S + 1), jnp.int32),
            jax.ShapeDtypeStruct((B, 1, 1), jnp.int32),
            jax.ShapeDtypeStruct((B, 1, 1), jnp.int32),
        ],
    )(target_logits, draft_probs, q, ids_sm.reshape(B, 1, S),
      ids_sm, u_sm, bonus_sm, greedy_sm)


def kernel(target_logits, draft_token_ids, bonus_token_ids, is_greedy,
           uniform_probs, q, draft_probs):
    B, S = draft_token_ids.shape
    idt = draft_token_ids.dtype

    out, nr, last = _fused(
        target_logits, draft_probs, q,
        draft_token_ids.astype(jnp.int32),
        uniform_probs,
        bonus_token_ids.reshape(1, B).astype(jnp.int32),
        is_greedy.reshape(1, B).astype(jnp.int32),
    )
    return (out.reshape(B, S + 1).astype(idt),
            nr.reshape(B).astype(jnp.int32),
            last.reshape(B).astype(idt))


# speedup vs baseline: 3.5875x; 1.8556x over previous
import jax
import jax.numpy as jnp
from jax import lax
from jax.experimental import pallas as pl
from jax.experimental.pallas import tpu as pltpu

_PLACEHOLDER = -1


def _stats_scan(target_logits, draft_probs, ids_sm, u_sm, bonus_sm,
                greedy_sm):
    B, S, V = target_logits.shape
    G = B // 2

    def body(lref, dp_hbm, ids_ref, u_ref, bon_ref, grd_ref,
             out_ref, nr_ref, la_ref, ss_ref, wr_ref, ms_ref, zs_ref,
             dchunk, dsem):
        i = pl.program_id(0)
        cur = lax.rem(i, 2)

        def chunk_copy(step, slot):
            cps = []
            for r2 in range(2):
                row = step * 2 + r2
                for s in range(S):
                    idx = ids_ref[row, s]
                    base = (idx // 128) * 128
                    cps.append(pltpu.make_async_copy(
                        dp_hbm.at[row, pl.ds(s, 1), pl.ds(base, 128)],
                        dchunk.at[slot, pl.ds(r2 * S + s, 1), :],
                        dsem.at[slot]))
            return cps

        @pl.when(i == 0)
        def _():
            for cp in chunk_copy(0, 0):
                cp.start()

        @pl.when(i + 1 < G)
        def _():
            for cp in chunk_copy(i + 1, lax.rem(i + 1, 2)):
                cp.start()

        for cp in chunk_copy(i, cur):
            cp.wait()

        Lv = jnp.concatenate([lref[0], lref[1]], axis=0)
        m8 = jnp.max(Lv, axis=1, keepdims=True)
        z8 = jnp.sum(jnp.exp(Lv - m8), axis=1, keepdims=True)
        iov = lax.broadcasted_iota(jnp.int32, Lv.shape, 1)
        am8 = jnp.min(jnp.where(Lv == m8, iov, V), axis=1, keepdims=True)

        io8 = lax.broadcasted_iota(jnp.int32, (2 * S, 1), 0)
        io128 = lax.broadcasted_iota(jnp.int32, (1, 128), 1)
        io128c = lax.broadcasted_iota(jnp.int32, (1, 1, 128), 2)
        io5 = lax.broadcasted_iota(jnp.int32, (1, 1, S + 1), 2)

        def fsel8(v, j):
            return jnp.sum(jnp.where(io8 == j, v, 0.0))

        def isel8(v, j):
            return jnp.sum(jnp.where(io8 == j, v, 0))

        for r2 in range(2):
            row = i * 2 + r2
            greedy = grd_ref[0, row] != 0
            bonus = bon_ref[0, row]
            prev_g = jnp.full((), True)
            prev_r = jnp.full((), True)
            numacc_g = jnp.full((), 0, jnp.int32)
            numacc_r = jnp.full((), 0, jnp.int32)
            toks = []
            ams = []
            matches = []
            ms = []
            zs = []
            for s in range(S):
                j = r2 * S + s
                m_s = fsel8(m8, j)
                z_s = fsel8(z8, j)
                am_s = isel8(am8, j)
                ms.append(m_s)
                zs.append(z_s)
                ams.append(am_s)
                idx = ids_ref[row, s]
                base = pl.multiple_of((idx // 128) * 128, 128)
                off = idx - base
                lchunk = lref[r2, pl.ds(s, 1), pl.ds(base, 128)]
                lat = jnp.sum(jnp.where(io128 == off, lchunk, 0.0))
                dchv = dchunk[pl.ds(cur, 1), pl.ds(j, 1), :]
                dat = jnp.sum(jnp.where(io128c == off, dchv, 0.0))
                t = jnp.exp(lat - m_s) / z_s
                acc_s = (dat > 0.0) & (
                    (t / jnp.where(dat > 0.0, dat, 1.0)) >= u_ref[row, s])
                match_s = idx == am_s
                matches.append(match_s)
                tok_g = jnp.where(prev_g, am_s, _PLACEHOLDER)
                tok_r = jnp.where(prev_r, jnp.where(acc_s, idx, 0),
                                  _PLACEHOLDER)
                toks.append(jnp.where(greedy, tok_g, tok_r))
                numacc_g += jnp.where(prev_g, 1, 0)
                numacc_r += jnp.where(prev_r, 1, 0)
                prev_g = prev_g & match_s
                prev_r = prev_r & acc_s
            all_g = prev_g
            all_r = prev_r
            numacc_g += jnp.where(all_g, 1, 0)
            numacc_r += jnp.where(all_r, 1, 0)
            tok_b = jnp.where((greedy & all_g) | ((~greedy) & all_r),
                              bonus, _PLACEHOLDER)

            numacc = jnp.where(greedy, numacc_g, numacc_r)
            first_rj = numacc_r - 1
            sstar = jnp.minimum(first_rj, S - 1)
            wr = (~greedy) & (first_rj < S)

            last_g = bonus
            for s in reversed(range(S)):
                last_g = jnp.where(matches[s], last_g, ams[s])
            last_nw = jnp.where(greedy, last_g, bonus)

            m_sel = ms[0]
            z_sel = zs[0]
            for s in range(1, S):
                pick = sstar == s
                m_sel = jnp.where(pick, ms[s], m_sel)
                z_sel = jnp.where(pick, zs[s], z_sel)

            rowv = jnp.broadcast_to(tok_b, (1, 1, S + 1))
            for s in reversed(range(S)):
                rowv = jnp.where(io5 == s, toks[s], rowv)
            out_ref[pl.ds(r2, 1)] = rowv
            nr_ref[pl.ds(r2, 1)] = jnp.broadcast_to(
                (S + 1) - numacc, (1, 1, 1))
            la_ref[pl.ds(r2, 1)] = jnp.broadcast_to(last_nw, (1, 1, 1))
            ss_ref[pl.ds(r2, 1)] = jnp.broadcast_to(sstar, (1, 1, 1))
            wr_ref[pl.ds(r2, 1)] = jnp.broadcast_to(
                jnp.where(wr, 1, 0), (1, 1, 1))
            ms_ref[pl.ds(r2, 1)] = jnp.broadcast_to(m_sel, (1, 1, 1))
            zs_ref[pl.ds(r2, 1)] = jnp.broadcast_to(z_sel, (1, 1, 1))

    specs_small = [pl.BlockSpec((2, 1, 1), lambda i: (i, 0, 0))] * 6
    shapes_small = [jax.ShapeDtypeStruct((B, 1, 1), d) for d in
                    (jnp.int32, jnp.int32, jnp.int32, jnp.int32,
                     jnp.float32, jnp.float32)]
    return pl.pallas_call(
        body,
        grid=(G,),
        in_specs=[
            pl.BlockSpec((2, S, V), lambda i: (i, 0, 0)),
            pl.BlockSpec(memory_space=pl.ANY),
            pl.BlockSpec(memory_space=pltpu.SMEM),
            pl.BlockSpec(memory_space=pltpu.SMEM),
            pl.BlockSpec(memory_space=pltpu.SMEM),
            pl.BlockSpec(memory_space=pltpu.SMEM),
        ],
        out_specs=[pl.BlockSpec((2, 1, S + 1), lambda i: (i, 0, 0))]
        + specs_small,
        out_shape=[jax.ShapeDtypeStruct((B, 1, S + 1), jnp.int32)]
        + shapes_small,
        scratch_shapes=[
            pltpu.VMEM((2, 2 * S, 128), jnp.float32),
            pltpu.SemaphoreType.DMA((2,)),
        ],
    )(target_logits, draft_probs, ids_sm, u_sm, bonus_sm, greedy_sm)


def _recover(sstar, target_logits, draft_probs, q, wr_sm, ms_sm, zs_sm,
             la_sm, outa):
    B, S, V = target_logits.shape
    NR = 8
    G = B // NR

    def body(ss_ref, l_hbm, dp_hbm, q_hbm, wr_ref, ms_ref, zs_ref, la_ref,
             oaref, out_ref, last_ref, lbuf, dbuf, qbuf, lsem, dsem, qsem):
        g = pl.program_id(0)
        cur = lax.rem(g, 2)

        def row_copies(step, slot):
            cps = []
            for k in range(NR):
                r = step * NR + k
                ssr = ss_ref[r]
                cps.append(pltpu.make_async_copy(
                    l_hbm.at[r, pl.ds(ssr, 1), :],
                    lbuf.at[slot, pl.ds(k, 1), :], lsem.at[slot]))
                cps.append(pltpu.make_async_copy(
                    dp_hbm.at[r, pl.ds(ssr, 1), :],
                    dbuf.at[slot, pl.ds(k, 1), :], dsem.at[slot]))
                cps.append(pltpu.make_async_copy(
                    q_hbm.at[pl.ds(r, 1), :],
                    qbuf.at[slot, pl.ds(k, 1), :], qsem.at[slot]))
            return cps

        @pl.when(g == 0)
        def _():
            for cp in row_copies(0, 0):
                cp.start()

        @pl.when(g + 1 < G)
        def _():
            for cp in row_copies(g + 1, lax.rem(g + 1, 2)):
                cp.start()

        for cp in row_copies(g, cur):
            cp.wait()

        io81 = lax.broadcasted_iota(jnp.int32, (1, NR, 1), 1)
        mv = jnp.zeros((1, NR, 1), jnp.float32)
        zv = jnp.ones((1, NR, 1), jnp.float32)
        for k in range(NR):
            r = g * NR + k
            mv = jnp.where(io81 == k, ms_ref[0, r], mv)
            zv = jnp.where(io81 == k, zs_ref[0, r], zv)

        Lv = lbuf[pl.ds(cur, 1)]
        Dv = dbuf[pl.ds(cur, 1)]
        Qv = qbuf[pl.ds(cur, 1)]
        p = jnp.exp(Lv - mv) / zv
        score = jnp.maximum(p - Dv, 0.0) * (1.0 / Qv)
        io3 = lax.broadcasted_iota(jnp.int32, score.shape, 2)
        smax = jnp.max(score, axis=2, keepdims=True)
        rec8 = jnp.min(jnp.where(score == smax, io3, V), axis=2)

        io18 = lax.broadcasted_iota(jnp.int32, (1, NR), 1)
        io5 = lax.broadcasted_iota(jnp.int32, (1, 1, S + 1), 2)
        for k in range(NR):
            r = g * NR + k
            rec_k = jnp.sum(jnp.where(io18 == k, rec8, 0))
            wrk = wr_ref[0, r] != 0
            ssk = ss_ref[r]
            partial = oaref[pl.ds(k, 1)]
            out_ref[pl.ds(k, 1)] = jnp.where(
                (io5 == ssk) & wrk, rec_k, partial)
            last_ref[pl.ds(k, 1)] = jnp.broadcast_to(
                jnp.where(wrk, rec_k, la_ref[0, r]), (1, 1, 1))

    grid_spec = pltpu.PrefetchScalarGridSpec(
        num_scalar_prefetch=1,
        grid=(G,),
        in_specs=[
            pl.BlockSpec(memory_space=pl.ANY),
            pl.BlockSpec(memory_space=pl.ANY),
            pl.BlockSpec(memory_space=pl.ANY),
            pl.BlockSpec(memory_space=pltpu.SMEM),
            pl.BlockSpec(memory_space=pltpu.SMEM),
            pl.BlockSpec(memory_space=pltpu.SMEM),
            pl.BlockSpec(memory_space=pltpu.SMEM),
            pl.BlockSpec((NR, 1, S + 1), lambda g, ss: (g, 0, 0)),
        ],
        out_specs=[
            pl.BlockSpec((NR, 1, S + 1), lambda g, ss: (g, 0, 0)),
            pl.BlockSpec((NR, 1, 1), lambda g, ss: (g, 0, 0)),
        ],
        scratch_shapes=[
            pltpu.VMEM((2, NR, V), jnp.float32),
            pltpu.VMEM((2, NR, V), jnp.float32),
            pltpu.VMEM((2, NR, V), jnp.float32),
            pltpu.SemaphoreType.DMA((2,)),
            pltpu.SemaphoreType.DMA((2,)),
            pltpu.SemaphoreType.DMA((2,)),
        ],
    )
    return pl.pallas_call(
        body,
        grid_spec=grid_spec,
        out_shape=[
            jax.ShapeDtypeStruct((B, 1, S + 1), jnp.int32),
            jax.ShapeDtypeStruct((B, 1, 1), jnp.int32),
        ],
    )(sstar, target_logits, draft_probs, q, wr_sm, ms_sm, zs_sm, la_sm,
      outa)


def kernel(target_logits, draft_token_ids, bonus_token_ids, is_greedy,
           uniform_probs, q, draft_probs):
    B, S = draft_token_ids.shape
    idt = draft_token_ids.dtype

    outa, nr, la, ss, wr, msel, zsel = _stats_scan(
        target_logits, draft_probs,
        draft_token_ids.astype(jnp.int32),
        uniform_probs,
        bonus_token_ids.reshape(1, B).astype(jnp.int32),
        is_greedy.reshape(1, B).astype(jnp.int32),
    )
    out, last = _recover(
        ss.reshape(B), target_logits, draft_probs, q,
        wr.reshape(1, B), msel.reshape(1, B), zsel.reshape(1, B),
        la.reshape(1, B), outa)
    return (out.reshape(B, S + 1).astype(idt),
            nr.reshape(B).astype(jnp.int32),
            last.reshape(B).astype(idt))


# speedup vs baseline: 6.2330x; 1.7374x over previous
import functools

import jax
import jax.numpy as jnp
from jax import lax
from jax.experimental import pallas as pl
from jax.experimental.pallas import tpu as pltpu
from jax.experimental.pallas import tpu_sc as plsc

_PLACEHOLDER = -1
_NEG_INF = float("-inf")


def _sc_gather(ids_flat, lflat, dflat, V, B):
    R = ids_flat.shape[0]
    info = plsc.get_sparse_core_info()
    nw = info.num_cores * info.num_subcores
    per = R // nw
    mesh = plsc.VectorSubcoreMesh(core_axis_name="c", subcore_axis_name="s")

    @functools.partial(
        pl.kernel,
        mesh=mesh,
        out_type=(
            jax.ShapeDtypeStruct((R,), jnp.float32),
            jax.ShapeDtypeStruct((R,), jnp.float32),
        ),
        scratch_types=[
            pltpu.VMEM((per,), jnp.int32),
            pltpu.VMEM((per,), jnp.int32),
            pltpu.VMEM((per,), jnp.float32),
            pltpu.VMEM((per,), jnp.float32),
            pltpu.SemaphoreType.DMA,
            pltpu.SemaphoreType.DMA,
        ],
    )
    def k(ids_hbm, lf_hbm, df_hbm, lat_hbm, dat_hbm,
          idx_v, flat_v, lat_v, dat_v, sem1, sem2):
        wid = lax.axis_index("s") * info.num_cores + lax.axis_index("c")
        base = wid * per
        pltpu.sync_copy(ids_hbm.at[pl.ds(base, per)], idx_v)
        w_per_s = B // per
        s_w = wid // w_per_s
        b0 = (wid - s_w * w_per_s) * per
        flat_v[...] = ((s_w * V + idx_v[...]) * B + b0
                       + lax.iota(jnp.int32, per))
        cp1 = pltpu.async_copy(lf_hbm.at[flat_v], lat_v, sem1)
        cp2 = pltpu.async_copy(df_hbm.at[flat_v], dat_v, sem2)
        cp1.wait()
        cp2.wait()
        pltpu.sync_copy(lat_v, lat_hbm.at[pl.ds(base, per)])
        pltpu.sync_copy(dat_v, dat_hbm.at[pl.ds(base, per)])

    return k(ids_flat, lflat, dflat)


def _k1_stats(tl_t, C):
    S, V, B = tl_t.shape
    NJ = V // C

    def body(lref, m_out, z_out, a_out, macc, zacc, aacc):
        j = pl.program_id(1)
        X = lref[0]
        lm = jnp.max(X, axis=0, keepdims=True)
        iov = lax.broadcasted_iota(jnp.int32, (C, B), 0) + j * C
        lam = jnp.min(jnp.where(X == lm, iov, V), axis=0, keepdims=True)

        @pl.when(j == 0)
        def _():
            macc[...] = lm
            zacc[...] = jnp.sum(jnp.exp(X - lm), axis=0, keepdims=True)
            aacc[...] = lam

        @pl.when(j > 0)
        def _():
            mo = macc[...]
            mn = jnp.maximum(mo, lm)
            zacc[...] = (zacc[...] * jnp.exp(mo - mn)
                         + jnp.sum(jnp.exp(X - mn), axis=0, keepdims=True))
            aacc[...] = jnp.where(lm > mo, lam, aacc[...])
            macc[...] = mn

        @pl.when(j == NJ - 1)
        def _():
            m_out[...] = macc[...][None]
            z_out[...] = zacc[...][None]
            a_out[...] = aacc[...][None]

    return pl.pallas_call(
        body,
        grid=(S, NJ),
        in_specs=[pl.BlockSpec((1, C, B), lambda s, j: (s, j, 0))],
        out_specs=[
            pl.BlockSpec((1, 1, B), lambda s, j: (s, 0, 0)),
            pl.BlockSpec((1, 1, B), lambda s, j: (s, 0, 0)),
            pl.BlockSpec((1, 1, B), lambda s, j: (s, 0, 0)),
        ],
        out_shape=[
            jax.ShapeDtypeStruct((S, 1, B), jnp.float32),
            jax.ShapeDtypeStruct((S, 1, B), jnp.float32),
            jax.ShapeDtypeStruct((S, 1, B), jnp.int32),
        ],
        scratch_shapes=[
            pltpu.VMEM((1, B), jnp.float32),
            pltpu.VMEM((1, B), jnp.float32),
            pltpu.VMEM((1, B), jnp.int32),
        ],
    )(tl_t)


def _scan(m_t, z_t, am_t, ids_t, lat_t, dat_t, u_t, bonus_r, greedy_r):
    S, B = ids_t.shape

    def body(mref, zref, aref, iref, lref, dref, uref, bref, gref,
             outref, nrref, laref, ssref, wrref, msref, zsref):
        m = mref[...]
        z = zref[...]
        am = aref[...]
        ids = iref[...]
        lat = lref[...]
        dat = dref[...]
        u = uref[...]
        bonus = bref[...]
        greedy = gref[...] != 0

        t = jnp.exp(lat - m) / z
        acc = (dat > 0.0) & ((t / jnp.where(dat > 0.0, dat, 1.0)) >= u)
        match = ids == am

        ones = jnp.ones((1, B), dtype=jnp.bool_)
        prev_g = ones
        prev_r = ones
        numacc_g = jnp.zeros((1, B), dtype=jnp.int32)
        numacc_r = jnp.zeros((1, B), dtype=jnp.int32)
        neg1 = jnp.full((1, B), _PLACEHOLDER, dtype=jnp.int32)
        for s in range(S):
            acc_s = acc[s:s + 1, :]
            match_s = match[s:s + 1, :]
            am_s = am[s:s + 1, :]
            ids_s = ids[s:s + 1, :]
            tok_g = jnp.where(prev_g, am_s, neg1)
            tok_r = jnp.where(prev_r, jnp.where(acc_s, ids_s, 0), neg1)
            outref[s:s + 1, :] = jnp.where(greedy, tok_g, tok_r)
            numacc_g += jnp.where(prev_g, 1, 0)
            numacc_r += jnp.where(prev_r, 1, 0)
            prev_g = prev_g & match_s
            prev_r = prev_r & acc_s
        numacc_g += jnp.where(prev_g, 1, 0)
        numacc_r += jnp.where(prev_r, 1, 0)
        ok_bonus = (greedy & prev_g) | ((~greedy) & prev_r)
        outref[S:S + 1, :] = jnp.where(ok_bonus, bonus, neg1)

        numacc = jnp.where(greedy, numacc_g, numacc_r)
        nrref[...] = (S + 1) - numacc

        first_rj = numacc_r - 1
        sstar = jnp.minimum(first_rj, S - 1)
        ssref[...] = sstar
        wrref[...] = jnp.where((~greedy) & (first_rj < S), 1, 0)

        last_g = bonus
        for s in reversed(range(S)):
            last_g = jnp.where(match[s:s + 1, :], last_g, am[s:s + 1, :])
        laref[...] = jnp.where(greedy, last_g, bonus)

        msel = m[0:1, :]
        zsel = z[0:1, :]
        for s in range(1, S):
            pick = sstar == s
            msel = jnp.where(pick, m[s:s + 1, :], msel)
            zsel = jnp.where(pick, z[s:s + 1, :], zsel)
        msref[...] = msel
        zsref[...] = zsel

    return pl.pallas_call(
        body,
        out_shape=[
            jax.ShapeDtypeStruct((S + 1, B), jnp.int32),
            jax.ShapeDtypeStruct((1, B), jnp.int32),
            jax.ShapeDtypeStruct((1, B), jnp.int32),
            jax.ShapeDtypeStruct((1, B), jnp.int32),
            jax.ShapeDtypeStruct((1, B), jnp.int32),
            jax.ShapeDtypeStruct((1, B), jnp.float32),
            jax.ShapeDtypeStruct((1, B), jnp.float32),
        ],
    )(m_t, z_t, am_t, ids_t, lat_t, dat_t, u_t, bonus_r, greedy_r)


def _k2_recover(tl_t, dp_t, q_t, sstar, wr, msel, zsel, outa, lastnw, C):
    S, V, B = tl_t.shape
    NJ = V // C

    def body(lref, dref, qref, ssref, wrref, msref, zsref, oaref, laref,
             out_ref, last_ref, gmax, gidx, gnan):
        j = pl.program_id(0)
        s = pl.program_id(1)

        @pl.when((j == 0) & (s == 0))
        def _():
            gmax[...] = jnp.full((1, B), _NEG_INF, jnp.float32)
            gidx[...] = jnp.zeros((1, B), jnp.int32)
            gnan[...] = jnp.full((1, B), V, jnp.int32)

        lanemask = (ssref[...] == s) & (wrref[...] != 0)
        X = lref[0]
        D = dref[0]
        Q = qref[...]
        p = jnp.exp(X - msref[...]) / zsref[...]
        sc = jnp.maximum(p - D, 0.0) * (1.0 / Q)
        iov = lax.broadcasted_iota(jnp.int32, (C, B), 0) + j * C
        nanm = sc != sc
        ln = jnp.min(jnp.where(nanm, iov, V), axis=0, keepdims=True)
        gnan[...] = jnp.minimum(gnan[...], jnp.where(lanemask, ln, V))
        scc = jnp.where(nanm, _NEG_INF, sc)
        lm = jnp.max(scc, axis=0, keepdims=True)
        lam = jnp.min(jnp.where(scc == lm, iov, V), axis=0, keepdims=True)
        upd = lanemask & (lm > gmax[...])
        gidx[...] = jnp.where(upd, lam, gidx[...])
        gmax[...] = jnp.where(upd, lm, gmax[...])

        @pl.when((j == NJ - 1) & (s == S - 1))
        def _():
            recn = gnan[...]
            rec = jnp.where(recn < V, recn, gidx[...])
            wrv = wrref[...] != 0
            oa = oaref[...]
            io = lax.broadcasted_iota(jnp.int32, (S + 1, B), 0)
            out_ref[...] = jnp.where((io == ssref[...]) & wrv, rec, oa)
            last_ref[...] = jnp.where(wrv, rec, laref[...])

    return pl.pallas_call(
        body,
        grid=(NJ, S),
        in_specs=[
            pl.BlockSpec((1, C, B), lambda j, s: (s, j, 0)),
            pl.BlockSpec((1, C, B), lambda j, s: (s, j, 0)),
            pl.BlockSpec((C, B), lambda j, s: (j, 0)),
            pl.BlockSpec((1, B), lambda j, s: (0, 0)),
            pl.BlockSpec((1, B), lambda j, s: (0, 0)),
            pl.BlockSpec((1, B), lambda j, s: (0, 0)),
            pl.BlockSpec((1, B), lambda j, s: (0, 0)),
            pl.BlockSpec((S + 1, B), lambda j, s: (0, 0)),
            pl.BlockSpec((1, B), lambda j, s: (0, 0)),
        ],
        out_specs=[
            pl.BlockSpec((S + 1, B), lambda j, s: (0, 0)),
            pl.BlockSpec((1, B), lambda j, s: (0, 0)),
        ],
        out_shape=[
            jax.ShapeDtypeStruct((S + 1, B), jnp.int32),
            jax.ShapeDtypeStruct((1, B), jnp.int32),
        ],
        scratch_shapes=[
            pltpu.VMEM((1, B), jnp.float32),
            pltpu.VMEM((1, B), jnp.int32),
            pltpu.VMEM((1, B), jnp.int32),
        ],
    )(tl_t, dp_t, q_t, sstar, wr, msel, zsel, outa, lastnw)


def kernel(target_logits, draft_token_ids, bonus_token_ids, is_greedy,
           uniform_probs, q, draft_probs):
    B, S = draft_token_ids.shape
    V = target_logits.shape[-1]
    idt = draft_token_ids.dtype
    C = 4000 if V % 4000 == 0 else V

    tl_t = jnp.transpose(target_logits, (1, 2, 0))
    dp_t = jnp.transpose(draft_probs, (1, 2, 0))
    q_t = jnp.transpose(q, (1, 0))

    ids_t = draft_token_ids.T.astype(jnp.int32)
    lat_f, dat_f = _sc_gather(
        ids_t.reshape(-1), tl_t.reshape(-1), dp_t.reshape(-1), V, B)

    m3, z3, a3 = _k1_stats(tl_t, C)

    outa, nr, lastnw, sstar, wr, msel, zsel = _scan(
        m3.reshape(S, B), z3.reshape(S, B), a3.reshape(S, B),
        ids_t, lat_f.reshape(S, B), dat_f.reshape(S, B),
        uniform_probs.T,
        bonus_token_ids.reshape(1, B).astype(jnp.int32),
        is_greedy.reshape(1, B).astype(jnp.int32),
    )

    out_t, last = _k2_recover(
        tl_t, dp_t, q_t, sstar, wr, msel, zsel, outa, lastnw, C)

    return (out_t.T.astype(idt),
            nr.reshape(B).astype(jnp.int32),
            last.reshape(B).astype(idt))


# speedup vs baseline: 6.2552x; 1.0036x over previous
import functools

import jax
import jax.numpy as jnp
from jax import lax
from jax.experimental import pallas as pl
from jax.experimental.pallas import tpu as pltpu
from jax.experimental.pallas import tpu_sc as plsc

_PLACEHOLDER = -1
_NEG_INF = float("-inf")


def _sc_gather(ids_flat, lflat, dflat, V, B):
    R = ids_flat.shape[0]
    info = plsc.get_sparse_core_info()
    nw = info.num_cores * info.num_subcores
    per = R // nw
    mesh = plsc.VectorSubcoreMesh(core_axis_name="c", subcore_axis_name="s")

    @functools.partial(
        pl.kernel,
        mesh=mesh,
        out_type=(
            jax.ShapeDtypeStruct((R,), jnp.float32),
            jax.ShapeDtypeStruct((R,), jnp.float32),
        ),
        scratch_types=[
            pltpu.VMEM((per,), jnp.int32),
            pltpu.VMEM((per,), jnp.int32),
            pltpu.VMEM((per,), jnp.float32),
            pltpu.VMEM((per,), jnp.float32),
            pltpu.SemaphoreType.DMA,
            pltpu.SemaphoreType.DMA,
        ],
    )
    def k(ids_hbm, lf_hbm, df_hbm, lat_hbm, dat_hbm,
          idx_v, flat_v, lat_v, dat_v, sem1, sem2):
        wid = lax.axis_index("s") * info.num_cores + lax.axis_index("c")
        base = wid * per
        pltpu.sync_copy(ids_hbm.at[pl.ds(base, per)], idx_v)
        w_per_s = B // per
        s_w = wid // w_per_s
        b0 = (wid - s_w * w_per_s) * per
        flat_v[...] = ((s_w * V + idx_v[...]) * B + b0
                       + lax.iota(jnp.int32, per))
        cp1 = pltpu.async_copy(lf_hbm.at[flat_v], lat_v, sem1)
        cp2 = pltpu.async_copy(df_hbm.at[flat_v], dat_v, sem2)
        cp1.wait()
        cp2.wait()
        pltpu.sync_copy(lat_v, lat_hbm.at[pl.ds(base, per)])
        pltpu.sync_copy(dat_v, dat_hbm.at[pl.ds(base, per)])

    return k(ids_flat, lflat, dflat)


def _k1_stats(tl_t, C):
    S, V, B = tl_t.shape
    NJ = V // C

    def body(lref, m_out, z_out, a_out, macc, zacc, aacc):
        j = pl.program_id(1)
        X = lref[0]
        lm = jnp.max(X, axis=0, keepdims=True)
        iov = lax.broadcasted_iota(jnp.int32, (C, B), 0) + j * C
        lam = jnp.min(jnp.where(X == lm, iov, V), axis=0, keepdims=True)

        @pl.when(j == 0)
        def _():
            macc[...] = lm
            zacc[...] = jnp.sum(jnp.exp(X - lm), axis=0, keepdims=True)
            aacc[...] = lam

        @pl.when(j > 0)
        def _():
            mo = macc[...]
            mn = jnp.maximum(mo, lm)
            zacc[...] = (zacc[...] * jnp.exp(mo - mn)
                         + jnp.sum(jnp.exp(X - mn), axis=0, keepdims=True))
            aacc[...] = jnp.where(lm > mo, lam, aacc[...])
            macc[...] = mn

        @pl.when(j == NJ - 1)
        def _():
            m_out[...] = macc[...][None]
            z_out[...] = zacc[...][None]
            a_out[...] = aacc[...][None]

    return pl.pallas_call(
        body,
        grid=(S, NJ),
        in_specs=[pl.BlockSpec((1, C, B), lambda s, j: (s, j, 0))],
        out_specs=[
            pl.BlockSpec((1, 1, B), lambda s, j: (s, 0, 0)),
            pl.BlockSpec((1, 1, B), lambda s, j: (s, 0, 0)),
            pl.BlockSpec((1, 1, B), lambda s, j: (s, 0, 0)),
        ],
        out_shape=[
            jax.ShapeDtypeStruct((S, 1, B), jnp.float32),
            jax.ShapeDtypeStruct((S, 1, B), jnp.float32),
            jax.ShapeDtypeStruct((S, 1, B), jnp.int32),
        ],
        scratch_shapes=[
            pltpu.VMEM((1, B), jnp.float32),
            pltpu.VMEM((1, B), jnp.float32),
            pltpu.VMEM((1, B), jnp.int32),
        ],
    )(tl_t)


def _scan(m_t, z_t, am_t, ids_t, lat_t, dat_t, u_t, bonus_r, greedy_r):
    S, B = ids_t.shape

    def body(mref, zref, aref, iref, lref, dref, uref, bref, gref,
             outref, nrref, laref, ssref, wrref, msref, zsref):
        m = mref[...]
        z = zref[...]
        am = aref[...]
        ids = iref[...]
        lat = lref[...]
        dat = dref[...]
        u = uref[...]
        bonus = bref[...]
        greedy = gref[...] != 0

        t = jnp.exp(lat - m) / z
        acc = (dat > 0.0) & ((t / jnp.where(dat > 0.0, dat, 1.0)) >= u)
        match = ids == am

        ones = jnp.ones((1, B), dtype=jnp.bool_)
        prev_g = ones
        prev_r = ones
        numacc_g = jnp.zeros((1, B), dtype=jnp.int32)
        numacc_r = jnp.zeros((1, B), dtype=jnp.int32)
        neg1 = jnp.full((1, B), _PLACEHOLDER, dtype=jnp.int32)
        for s in range(S):
            acc_s = acc[s:s + 1, :]
            match_s = match[s:s + 1, :]
            am_s = am[s:s + 1, :]
            ids_s = ids[s:s + 1, :]
            tok_g = jnp.where(prev_g, am_s, neg1)
            tok_r = jnp.where(prev_r, jnp.where(acc_s, ids_s, 0), neg1)
            outref[s:s + 1, :] = jnp.where(greedy, tok_g, tok_r)
            numacc_g += jnp.where(prev_g, 1, 0)
            numacc_r += jnp.where(prev_r, 1, 0)
            prev_g = prev_g & match_s
            prev_r = prev_r & acc_s
        numacc_g += jnp.where(prev_g, 1, 0)
        numacc_r += jnp.where(prev_r, 1, 0)
        ok_bonus = (greedy & prev_g) | ((~greedy) & prev_r)
        outref[S:S + 1, :] = jnp.where(ok_bonus, bonus, neg1)

        numacc = jnp.where(greedy, numacc_g, numacc_r)
        nrref[...] = (S + 1) - numacc

        first_rj = numacc_r - 1
        sstar = jnp.minimum(first_rj, S - 1)
        ssref[...] = sstar
        wrref[...] = jnp.where((~greedy) & (first_rj < S), 1, 0)

        last_g = bonus
        for s in reversed(range(S)):
            last_g = jnp.where(match[s:s + 1, :], last_g, am[s:s + 1, :])
        laref[...] = jnp.where(greedy, last_g, bonus)

        msel = m[0:1, :]
        zsel = z[0:1, :]
        for s in range(1, S):
            pick = sstar == s
            msel = jnp.where(pick, m[s:s + 1, :], msel)
            zsel = jnp.where(pick, z[s:s + 1, :], zsel)
        msref[...] = msel
        zsref[...] = zsel

    return pl.pallas_call(
        body,
        out_shape=[
            jax.ShapeDtypeStruct((S + 1, B), jnp.int32),
            jax.ShapeDtypeStruct((1, B), jnp.int32),
            jax.ShapeDtypeStruct((1, B), jnp.int32),
            jax.ShapeDtypeStruct((1, B), jnp.int32),
            jax.ShapeDtypeStruct((1, B), jnp.int32),
            jax.ShapeDtypeStruct((1, B), jnp.float32),
            jax.ShapeDtypeStruct((1, B), jnp.float32),
        ],
    )(m_t, z_t, am_t, ids_t, lat_t, dat_t, u_t, bonus_r, greedy_r)


def _k2_recover(tl_t, dp_t, q_t, sstar, wr, msel, zsel, outa, lastnw, C):
    S, V, B = tl_t.shape
    NJ = V // C

    def body(lref, dref, qref, ssref, wrref, msref, zsref, oaref, laref,
             out_ref, last_ref, gmax, gidx, gnan):
        j = pl.program_id(0)
        s = pl.program_id(1)

        @pl.when((j == 0) & (s == 0))
        def _():
            gmax[...] = jnp.full((1, B), _NEG_INF, jnp.float32)
            gidx[...] = jnp.zeros((1, B), jnp.int32)
            gnan[...] = jnp.full((1, B), V, jnp.int32)

        lanemask = (ssref[...] == s) & (wrref[...] != 0)
        X = lref[0]
        D = dref[0]
        Q = qref[...]
        p = jnp.exp(X - msref[...]) / zsref[...]
        sc = jnp.maximum(p - D, 0.0) * (1.0 / Q)
        iov = lax.broadcasted_iota(jnp.int32, (C, B), 0) + j * C
        nanm = sc != sc
        ln = jnp.min(jnp.where(nanm, iov, V), axis=0, keepdims=True)
        gnan[...] = jnp.minimum(gnan[...], jnp.where(lanemask, ln, V))
        scc = jnp.where(nanm, _NEG_INF, sc)
        lm = jnp.max(scc, axis=0, keepdims=True)
        lam = jnp.min(jnp.where(scc == lm, iov, V), axis=0, keepdims=True)
        upd = lanemask & (lm > gmax[...])
        gidx[...] = jnp.where(upd, lam, gidx[...])
        gmax[...] = jnp.where(upd, lm, gmax[...])

        @pl.when((j == NJ - 1) & (s == S - 1))
        def _():
            recn = gnan[...]
            rec = jnp.where(recn < V, recn, gidx[...])
            wrv = wrref[...] != 0
            oa = oaref[...]
            io = lax.broadcasted_iota(jnp.int32, (S + 1, B), 0)
            out_ref[...] = jnp.where((io == ssref[...]) & wrv, rec, oa)
            last_ref[...] = jnp.where(wrv, rec, laref[...])

    return pl.pallas_call(
        body,
        grid=(NJ, S),
        in_specs=[
            pl.BlockSpec((1, C, B), lambda j, s: (s, j, 0)),
            pl.BlockSpec((1, C, B), lambda j, s: (s, j, 0)),
            pl.BlockSpec((C, B), lambda j, s: (j, 0)),
            pl.BlockSpec((1, B), lambda j, s: (0, 0)),
            pl.BlockSpec((1, B), lambda j, s: (0, 0)),
            pl.BlockSpec((1, B), lambda j, s: (0, 0)),
            pl.BlockSpec((1, B), lambda j, s: (0, 0)),
            pl.BlockSpec((S + 1, B), lambda j, s: (0, 0)),
            pl.BlockSpec((1, B), lambda j, s: (0, 0)),
        ],
        out_specs=[
            pl.BlockSpec((S + 1, B), lambda j, s: (0, 0)),
            pl.BlockSpec((1, B), lambda j, s: (0, 0)),
        ],
        out_shape=[
            jax.ShapeDtypeStruct((S + 1, B), jnp.int32),
            jax.ShapeDtypeStruct((1, B), jnp.int32),
        ],
        scratch_shapes=[
            pltpu.VMEM((1, B), jnp.float32),
            pltpu.VMEM((1, B), jnp.int32),
            pltpu.VMEM((1, B), jnp.int32),
        ],
    )(tl_t, dp_t, q_t, sstar, wr, msel, zsel, outa, lastnw)


def kernel(target_logits, draft_token_ids, bonus_token_ids, is_greedy,
           uniform_probs, q, draft_probs):
    B, S = draft_token_ids.shape
    V = target_logits.shape[-1]
    idt = draft_token_ids.dtype
    C = 10000 if V % 10000 == 0 else V

    tl_t = jnp.transpose(target_logits, (1, 2, 0))
    dp_t = jnp.transpose(draft_probs, (1, 2, 0))
    q_t = jnp.transpose(q, (1, 0))

    ids_t = draft_token_ids.T.astype(jnp.int32)
    lat_f, dat_f = _sc_gather(
        ids_t.reshape(-1), tl_t.reshape(-1), dp_t.reshape(-1), V, B)

    m3, z3, a3 = _k1_stats(tl_t, C)

    outa, nr, lastnw, sstar, wr, msel, zsel = _scan(
        m3.reshape(S, B), z3.reshape(S, B), a3.reshape(S, B),
        ids_t, lat_f.reshape(S, B), dat_f.reshape(S, B),
        uniform_probs.T,
        bonus_token_ids.reshape(1, B).astype(jnp.int32),
        is_greedy.reshape(1, B).astype(jnp.int32),
    )

    out_t, last = _k2_recover(
        tl_t, dp_t, q_t, sstar, wr, msel, zsel, outa, lastnw, C)

    return (out_t.T.astype(idt),
            nr.reshape(B).astype(jnp.int32),
            last.reshape(B).astype(idt))
